# baseline (speedup 1.0000x reference)
import jax
import jax.numpy as jnp
from jax import lax
from jax.experimental import pallas as pl
from jax.experimental.pallas import tpu as pltpu
from jax.experimental.pallas import tpu_sc as plsc

N = 10000
E = 320000
IN_CH = 128
H = 8
C = 16
HC = H * C

NC = 2
NS = 16
NH = N // NC
AROWS = 5120
GROW = NH
EPW = E // NS
CH = 32
NG = CH // 16
NCHUNK = EPW // CH
ZROWS = CH
NZ = AROWS // ZROWS // NS
HP = H // 2
PDW = NH * HP + 64
HIMASK = -65536


def _pack(hi_f32, lo_f32):
    hi = lax.bitcast_convert_type(hi_f32.astype(jnp.bfloat16),
                                  jnp.uint16).astype(jnp.uint32)
    lo = lax.bitcast_convert_type(lo_f32.astype(jnp.bfloat16),
                                  jnp.uint16).astype(jnp.uint32)
    return lax.bitcast_convert_type((hi << 16) | lo, jnp.int32)


def _proj_body(x_ref, w_ref, a_ref, a2_ref, h_ref, sd_ref, pd_ref):
    xb = x_ref[...]
    hb = jnp.dot(xb, w_ref[...], preferred_element_type=jnp.float32)
    h_ref[...] = hb
    sd_ref[...] = jnp.dot(hb, a_ref[...], preferred_element_type=jnp.float32)
    sd2 = jnp.dot(hb, a2_ref[...], preferred_element_type=jnp.float32)
    pd_ref[...] = _pack(sd2[:, 0:HP], sd2[:, HP:2 * HP])


def _project(x, w, a, a2, bn):
    grid = N // bn
    return pl.pallas_call(
        _proj_body,
        grid=(grid,),
        in_specs=[
            pl.BlockSpec((bn, IN_CH), lambda i: (i, 0)),
            pl.BlockSpec((IN_CH, HC), lambda i: (0, 0)),
            pl.BlockSpec((HC, HC), lambda i: (0, 0)),
            pl.BlockSpec((HC, H), lambda i: (0, 0)),
        ],
        out_specs=[
            pl.BlockSpec((bn, HC), lambda i: (i, 0)),
            pl.BlockSpec((bn, HC), lambda i: (i, 0)),
            pl.BlockSpec((bn, HP), lambda i: (i, 0)),
        ],
        out_shape=[
            jax.ShapeDtypeStruct((N, HC), jnp.float32),
            jax.ShapeDtypeStruct((N, HC), jnp.float32),
            jax.ShapeDtypeStruct((N, HP), jnp.int32),
        ],
    )(x, w, a, a2)


def _sc_body(src_hbm, dst_hbm, h_hbm, sd_hbm, pd_hbm, acc_hbm, den_hbm,
             sidx0, sidx1, didx0, didx1, hrow0, hrow1, sds0, sds1,
             msg, wbuf, adbuf, pdvm, den, acc, sem0, sem1):
    sidx = (sidx0, sidx1)
    didx = (didx0, didx1)
    hrow = (hrow0, hrow1)
    sds = (sds0, sds1)
    sem = (sem0, sem1)
    izeros = jnp.zeros((16,), jnp.int32)
    c = lax.axis_index("c")
    s = lax.axis_index("s")
    wid = c * NS + s
    dbase = c * NH

    zeros = jnp.zeros((16,), jnp.float32)

    PS_P = 2000
    for k in range(NH * HP // PS_P):
        pltpu.sync_copy(pd_hbm.at[pl.ds(c * (NH * HP) + k * PS_P, PS_P)],
                        pdvm.at[pl.ds(k * PS_P, PS_P)])
    for j in range(64 // 16):
        pdvm[pl.ds(NH * HP + j * 16, 16)] = izeros

    def _zden(i, _):
        den[pl.ds(i * 16, 16)] = zeros
        return 0
    lax.fori_loop(0, AROWS * H // 16, _zden, 0)

    def _zrow(i, _):
        for j in range(HC // 16):
            msg[i, pl.ds(j * 16, 16)] = zeros
        return 0
    lax.fori_loop(0, ZROWS, _zrow, 0)

    def _zchunk(jj, _):
        off = (jj * NS + s) * ZROWS
        pltpu.sync_copy(msg, acc.at[pl.ds(off, ZROWS)])
        return 0
    lax.fori_loop(0, NZ, _zchunk, 0)

    plsc.subcore_barrier()

    lanes = lax.iota(jnp.int32, 16)

    def _fire(i, b):
        base = s * EPW + i * CH
        pltpu.sync_copy(src_hbm.at[pl.ds(base, CH)], sidx[b])
        pltpu.sync_copy(dst_hbm.at[pl.ds(base, CH)], didx[b])
        pltpu.async_copy(h_hbm.at[sidx[b]], hrow[b], sem[b])
        pltpu.async_copy(sd_hbm.at[sidx[b]], sds[b], sem[b])

    def _make_edge(b):
        hrow_b, sds_b = hrow[b], sds[b]

        def _edge(e, _):
            sv = sds_b[e, pl.ds(0, 16)]
            dv = adbuf[pl.ds(e * 16, 16)]
            a = sv + dv
            a = jnp.where(a >= 0.0, a, 0.2 * a)
            w = jnp.exp(a)
            wbuf[pl.ds(e * 16, 16)] = w
            for h in range(H):
                msg[e, pl.ds(h * C, 16)] = w[h] * hrow_b[e, pl.ds(h * C, 16)]
            return 0
        return _edge

    _edges = (_make_edge(0), _make_edge(1))

    _fire(0, 0)
    _fire(1, 1)

    def _process(b):
        for g in range(NG):
            dvec = didx[b][pl.ds(g * 16, 16)]
            dl = dvec - dbase
            bad = (dl < 0) | (dl >= NH)
            dl = jnp.where(bad, GROW, dl)
            didx[b][pl.ds(g * 16, 16)] = dl
            evec = lanes + g * 16
            for hp in range(HP):
                pdw = plsc.load_gather(pdvm, [dl * HP + hp])
                plsc.store_scatter(
                    adbuf, [evec * 16 + 2 * hp],
                    plsc.bitcast(pdw & HIMASK, jnp.float32))
                plsc.store_scatter(
                    adbuf, [evec * 16 + 2 * hp + 1],
                    plsc.bitcast(lax.shift_left(pdw, 16), jnp.float32))
        pltpu.make_async_copy(h_hbm.at[sidx[b]], hrow[b], sem[b]).wait()
        pltpu.make_async_copy(sd_hbm.at[sidx[b]], sds[b], sem[b]).wait()
        lax.fori_loop(0, CH, _edges[b], 0)
        pltpu.sync_copy(msg, acc.at[didx[b]], add=True)
        for g in range(NG):
            dvec = didx[b][pl.ds(g * 16, 16)]
            widx = lanes * 16 + g * 256
            for h in range(H):
                wv = plsc.load_gather(wbuf, [widx + h])
                plsc.addupdate_scatter(den, [dvec * H + h], wv)

    def _body(j, _):
        for b in range(2):
            i = 2 * j + b
            _process(b)

            @pl.when(i + 2 < NCHUNK)
            def _():
                _fire(i + 2, b)
        return 0

    lax.fori_loop(0, NCHUNK // 2, _body, 0)
    _process(0)

    DP = AROWS * H // 8
    for k in range(8):
        pltpu.sync_copy(den.at[pl.ds(k * DP, DP)],
                        den_hbm.at[wid, pl.ds(k * DP, DP)])

    plsc.subcore_barrier()

    def _ochunk(jj, _):
        off = (jj * NS + s) * ZROWS
        pltpu.sync_copy(acc.at[pl.ds(off, ZROWS)], msg)
        pltpu.sync_copy(msg, acc_hbm.at[c, pl.ds(off, ZROWS)])
        return 0
    lax.fori_loop(0, NZ, _ochunk, 0)


def _sc_edges(src, dst, h, sd, pd):
    mesh = plsc.VectorSubcoreMesh(core_axis_name="c", subcore_axis_name="s",
                                  num_cores=NC)
    k = pl.kernel(
        _sc_body,
        out_type=[
            jax.ShapeDtypeStruct((NC, AROWS, HC), jnp.float32),
            jax.ShapeDtypeStruct((NC * NS, AROWS * H), jnp.float32),
        ],
        mesh=mesh,
        compiler_params=pltpu.CompilerParams(needs_layout_passes=False),
        scratch_types=[
            pltpu.VMEM((CH,), jnp.int32),
            pltpu.VMEM((CH,), jnp.int32),
            pltpu.VMEM((CH,), jnp.int32),
            pltpu.VMEM((CH,), jnp.int32),
            pltpu.VMEM((CH, HC), jnp.float32),
            pltpu.VMEM((CH, HC), jnp.float32),
            pltpu.VMEM((CH, HC), jnp.float32),
            pltpu.VMEM((CH, HC), jnp.float32),
            pltpu.VMEM((CH, HC), jnp.float32),
            pltpu.VMEM((CH * 16,), jnp.float32),
            pltpu.VMEM((CH * 16,), jnp.float32),
            pltpu.VMEM((PDW,), jnp.int32),
            pltpu.VMEM((AROWS * H,), jnp.float32),
            pltpu.VMEM_SHARED((AROWS, HC), jnp.float32),
            pltpu.SemaphoreType.DMA,
            pltpu.SemaphoreType.DMA,
        ],
    )
    return k(src, dst, h, sd, pd)


BN = 1000
BPH = NH // BN


def _comb_body(p_ref, dp_ref, h_ref, sd_ref, b_ref, r_ref, o_ref):
    a = p_ref[0]
    sd = sd_ref[...]
    al = sd[:, 0:H] + sd[:, 16:16 + H]
    al = jnp.where(al >= 0.0, al, 0.2 * al)
    ws = jnp.exp(al)
    r = r_ref[...]
    ws_bc = jnp.dot(ws, r, preferred_element_type=jnp.float32)
    den = jnp.sum(dp_ref[0], axis=0) + ws
    den_bc = jnp.dot(den, r, preferred_element_type=jnp.float32) + 1e-16
    o_ref[...] = (a + ws_bc * h_ref[...]) / den_bc + b_ref[...]


def _combine(p, dp, h, sd, bias, r):
    grid = N // BN
    return pl.pallas_call(
        _comb_body,
        grid=(grid,),
        in_specs=[
            pl.BlockSpec((1, BN, HC), lambda i: (i // BPH, i % BPH, 0)),
            pl.BlockSpec((1, NS, BN, H), lambda i: (i // BPH, 0, i % BPH, 0)),
            pl.BlockSpec((BN, HC), lambda i: (i, 0)),
            pl.BlockSpec((BN, HC), lambda i: (i, 0)),
            pl.BlockSpec((1, HC), lambda i: (0, 0)),
            pl.BlockSpec((H, HC), lambda i: (0, 0)),
        ],
        out_specs=pl.BlockSpec((BN, HC), lambda i: (i, 0)),
        out_shape=jax.ShapeDtypeStruct((N, HC), jnp.float32),
    )(p, dp, h, sd, bias, r)


def kernel(x, edge_index, W, att_src, att_dst, bias):
    src = edge_index[0]
    dst = edge_index[1]

    eye = jnp.eye(H, dtype=jnp.float32)
    asrc = jnp.reshape(eye[:, None, :] * att_src.reshape(H, C)[:, :, None],
                       (HC, H))
    adst = jnp.reshape(eye[:, None, :] * att_dst.reshape(H, C)[:, :, None],
                       (HC, H))
    amat = jnp.concatenate(
        [asrc, asrc, adst, adst,
         jnp.zeros((HC, HC - 4 * H), jnp.float32)], axis=1)

    a2mat = jnp.concatenate([adst[:, 0::2], adst[:, 1::2]], axis=1)
    h, sd, pd = _project(x, W, amat, a2mat, bn=1000)
    partial, denp = _sc_edges(src, dst, h, sd, pd.reshape(N * HP))

    r = jnp.repeat(jnp.eye(H, dtype=jnp.float32), C, axis=1)
    out = _combine(partial, denp.reshape(NC, NS, AROWS, H), h, sd,
                   bias.reshape(1, HC), r)
    return out


# speedup vs baseline: 2.0536x; 2.0536x over previous
import jax
import jax.numpy as jnp
from jax import lax
from jax.experimental import pallas as pl
from jax.experimental.pallas import tpu as pltpu
from jax.experimental.pallas import tpu_sc as plsc

N = 10000
E = 320000
IN_CH = 128
H = 8
C = 16
HC = H * C

NC = 2
NS = 16
NH = N // NC
AROWS = 5120
GROW = NH
EPW = E // NS
CH = 32
NG = CH // 16
NCHUNK = EPW // CH
ZROWS = CH
NZ = AROWS // ZROWS // NS
HP = H // 2
PDW = NH * HP + 64
HIMASK = -65536


def _pack(hi_f32, lo_f32):
    hi = lax.bitcast_convert_type(hi_f32.astype(jnp.bfloat16),
                                  jnp.uint16).astype(jnp.uint32)
    lo = lax.bitcast_convert_type(lo_f32.astype(jnp.bfloat16),
                                  jnp.uint16).astype(jnp.uint32)
    return lax.bitcast_convert_type((hi << 16) | lo, jnp.int32)


def _proj_body(x_ref, w_ref, a_ref, a2_ref, h_ref, sd_ref, pd_ref):
    xb = x_ref[...]
    hb = jnp.dot(xb, w_ref[...], preferred_element_type=jnp.float32)
    h_ref[...] = hb
    sd_ref[...] = jnp.dot(hb, a_ref[...], preferred_element_type=jnp.float32)
    sd2 = jnp.dot(hb, a2_ref[...], preferred_element_type=jnp.float32)
    pd_ref[...] = _pack(sd2[:, 0:HP], sd2[:, HP:2 * HP])


def _project(x, w, a, a2, bn):
    grid = N // bn
    return pl.pallas_call(
        _proj_body,
        grid=(grid,),
        in_specs=[
            pl.BlockSpec((bn, IN_CH), lambda i: (i, 0)),
            pl.BlockSpec((IN_CH, HC), lambda i: (0, 0)),
            pl.BlockSpec((HC, HC), lambda i: (0, 0)),
            pl.BlockSpec((HC, H), lambda i: (0, 0)),
        ],
        out_specs=[
            pl.BlockSpec((bn, HC), lambda i: (i, 0)),
            pl.BlockSpec((bn, HC), lambda i: (i, 0)),
            pl.BlockSpec((bn, HP), lambda i: (i, 0)),
        ],
        out_shape=[
            jax.ShapeDtypeStruct((N, HC), jnp.float32),
            jax.ShapeDtypeStruct((N, HC), jnp.float32),
            jax.ShapeDtypeStruct((N, HP), jnp.int32),
        ],
    )(x, w, a, a2)


AC = 400
NR = 5
REDGE = EPW // NR
NACHUNK = REDGE // AC
CMAX = REDGE + 4 * CH


def _sc_body(src_hbm, dst_hbm, h_hbm, sd_hbm, pd_hbm, acc_hbm, den_hbm,
             sidxa0, sidxa1, didxa0, didxa1, cpk, sidxb0, sidxb1,
             didxb0, didxb1, hrow0, hrow1, sds0, sds1,
             msg, wbuf, adbuf, pdvm, den, acc, sem0, sem1):
    sidxa = (sidxa0, sidxa1)
    didxa = (didxa0, didxa1)
    sidxb = (sidxb0, sidxb1)
    didxb = (didxb0, didxb1)
    hrow = (hrow0, hrow1)
    sds = (sds0, sds1)
    sem = (sem0, sem1)

    c = lax.axis_index("c")
    s = lax.axis_index("s")
    wid = c * NS + s
    dbase = c * NH

    zeros = jnp.zeros((16,), jnp.float32)
    izeros = jnp.zeros((16,), jnp.int32)
    lanes = lax.iota(jnp.int32, 16)

    PS_P = 2000
    for k in range(NH * HP // PS_P):
        pltpu.sync_copy(pd_hbm.at[pl.ds(c * (NH * HP) + k * PS_P, PS_P)],
                        pdvm.at[pl.ds(k * PS_P, PS_P)])
    for j in range(64 // 16):
        pdvm[pl.ds(NH * HP + j * 16, 16)] = izeros

    def _zden(i, _):
        den[pl.ds(i * 16, 16)] = zeros
        return 0
    lax.fori_loop(0, AROWS * H // 16, _zden, 0)

    def _zrow(i, _):
        for j in range(HC // 16):
            msg[i, pl.ds(j * 16, 16)] = zeros
        return 0
    lax.fori_loop(0, ZROWS, _zrow, 0)

    def _zchunk(jj, _):
        off = (jj * NS + s) * ZROWS
        pltpu.sync_copy(msg, acc.at[pl.ds(off, ZROWS)])
        return 0
    lax.fori_loop(0, NZ, _zchunk, 0)

    plsc.subcore_barrier()

    def _fire_a(rbase, i, b):
        base = rbase + i * AC
        pltpu.async_copy(src_hbm.at[pl.ds(base, AC)], sidxa[b], sem[b])
        pltpu.async_copy(dst_hbm.at[pl.ds(base, AC)], didxa[b], sem[b])

    def _process_a(b, cursor):
        pltpu.make_async_copy(src_hbm.at[pl.ds(0, AC)], sidxa[b],
                              sem[b]).wait()
        pltpu.make_async_copy(dst_hbm.at[pl.ds(0, AC)], didxa[b],
                              sem[b]).wait()
        for g in range(AC // 16):
            svec = sidxa[b][pl.ds(g * 16, 16)]
            dvec = didxa[b][pl.ds(g * 16, 16)]
            dl = dvec - dbase
            own = (dl >= 0) & (dl < NH)
            pos = cursor + plsc.cumsum(own.astype(jnp.int32)) - 1
            packed = lax.shift_left(svec, 16) | dl
            plsc.store_scatter(cpk, [pos], packed, mask=own)
            cursor = pos[15] + 1
        return cursor

    def _fire_b(i, b):
        base = i * CH
        for g in range(NG):
            w = plsc.load_gather(cpk, [base + g * 16 + lanes])
            sidxb[b][pl.ds(g * 16, 16)] = lax.shift_right_logical(w, 16)
            didxb[b][pl.ds(g * 16, 16)] = w & 0xFFFF
        pltpu.async_copy(h_hbm.at[sidxb[b]], hrow[b], sem[b])
        pltpu.async_copy(sd_hbm.at[sidxb[b]], sds[b], sem[b])

    def _make_edge(b):
        hrow_b, sds_b = hrow[b], sds[b]

        def _edge(e, _):
            sv = sds_b[e, pl.ds(0, 16)]
            dv = adbuf[pl.ds(e * 16, 16)]
            a = sv + dv
            a = jnp.where(a >= 0.0, a, 0.2 * a)
            w = jnp.exp(a)
            wbuf[pl.ds(e * 16, 16)] = w
            for h in range(H):
                msg[e, pl.ds(h * C, 16)] = w[h] * hrow_b[e, pl.ds(h * C, 16)]
            return 0
        return _edge

    _edges = (_make_edge(0), _make_edge(1))

    def _process_b(b):
        for g in range(NG):
            dl = didxb[b][pl.ds(g * 16, 16)]
            evec = lanes + g * 16
            for hp in range(HP):
                pdw = plsc.load_gather(pdvm, [dl * HP + hp])
                plsc.store_scatter(
                    adbuf, [evec * 16 + 2 * hp],
                    plsc.bitcast(pdw & HIMASK, jnp.float32))
                plsc.store_scatter(
                    adbuf, [evec * 16 + 2 * hp + 1],
                    plsc.bitcast(lax.shift_left(pdw, 16), jnp.float32))
        pltpu.make_async_copy(h_hbm.at[sidxb[b]], hrow[b], sem[b]).wait()
        pltpu.make_async_copy(sd_hbm.at[sidxb[b]], sds[b], sem[b]).wait()
        lax.fori_loop(0, CH, _edges[b], 0)
        pltpu.sync_copy(msg, acc.at[didxb[b]], add=True)
        for g in range(NG):
            dvec = didxb[b][pl.ds(g * 16, 16)]
            widx = lanes * 16 + g * 256
            for h in range(H):
                wv = plsc.load_gather(wbuf, [widx + h])
                plsc.addupdate_scatter(den, [dvec * H + h], wv)

    dummy = jnp.full((16,), GROW, jnp.int32)

    def _round(r, _):
        rbase = s * EPW + r * REDGE

        _fire_a(rbase, 0, 0)
        _fire_a(rbase, 1, 1)

        def _body_a(j, cursor):
            for b in range(2):
                i = 2 * j + b
                cursor = _process_a(b, cursor)

                @pl.when(i + 2 < NACHUNK)
                def _():
                    _fire_a(rbase, i + 2, b)
            return cursor

        m = lax.fori_loop(0, NACHUNK // 2, _body_a, jnp.int32(0))

        for g in range(4 * CH // 16):
            plsc.store_scatter(cpk, [m + lanes + g * 16], dummy)
        mpad = lax.max(((m + 2 * CH - 1) // (2 * CH)) * (2 * CH),
                       jnp.int32(2 * CH))
        nb = mpad // (2 * CH)

        _fire_b(0, 0)
        _fire_b(1, 1)

        def _body_b(j, _2):
            for b in range(2):
                i = 2 * j + b
                _process_b(b)

                @pl.when(i + 2 < 2 * nb)
                def _():
                    _fire_b(i + 2, b)
            return 0

        lax.fori_loop(0, nb, _body_b, 0)
        return 0

    lax.fori_loop(0, NR, _round, 0)

    DP = AROWS * H // 8
    for k in range(8):
        pltpu.sync_copy(den.at[pl.ds(k * DP, DP)],
                        den_hbm.at[wid, pl.ds(k * DP, DP)])

    plsc.subcore_barrier()

    def _ochunk(jj, _):
        off = (jj * NS + s) * ZROWS
        pltpu.sync_copy(acc.at[pl.ds(off, ZROWS)], msg)
        pltpu.sync_copy(msg, acc_hbm.at[c, pl.ds(off, ZROWS)])
        return 0
    lax.fori_loop(0, NZ, _ochunk, 0)


def _sc_edges(src, dst, h, sd, pd):
    mesh = plsc.VectorSubcoreMesh(core_axis_name="c", subcore_axis_name="s",
                                  num_cores=NC)
    k = pl.kernel(
        _sc_body,
        out_type=[
            jax.ShapeDtypeStruct((NC, AROWS, HC), jnp.float32),
            jax.ShapeDtypeStruct((NC * NS, AROWS * H), jnp.float32),
        ],
        mesh=mesh,
        compiler_params=pltpu.CompilerParams(needs_layout_passes=False),
        scratch_types=[
            pltpu.VMEM((AC,), jnp.int32),
            pltpu.VMEM((AC,), jnp.int32),
            pltpu.VMEM((AC,), jnp.int32),
            pltpu.VMEM((AC,), jnp.int32),
            pltpu.VMEM((CMAX,), jnp.int32),
            pltpu.VMEM((CH,), jnp.int32),
            pltpu.VMEM((CH,), jnp.int32),
            pltpu.VMEM((CH,), jnp.int32),
            pltpu.VMEM((CH,), jnp.int32),
            pltpu.VMEM((CH, HC), jnp.float32),
            pltpu.VMEM((CH, HC), jnp.float32),
            pltpu.VMEM((CH, HC), jnp.float32),
            pltpu.VMEM((CH, HC), jnp.float32),
            pltpu.VMEM((CH, HC), jnp.float32),
            pltpu.VMEM((CH * 16,), jnp.float32),
            pltpu.VMEM((CH * 16,), jnp.float32),
            pltpu.VMEM((PDW,), jnp.int32),
            pltpu.VMEM((AROWS * H,), jnp.float32),
            pltpu.VMEM_SHARED((AROWS, HC), jnp.float32),
            pltpu.SemaphoreType.DMA,
            pltpu.SemaphoreType.DMA,
        ],
    )
    return k(src, dst, h, sd, pd)


BN = 1000
BPH = NH // BN


def _comb_body(p_ref, dp_ref, h_ref, sd_ref, b_ref, r_ref, o_ref):
    a = p_ref[0]
    sd = sd_ref[...]
    al = sd[:, 0:H] + sd[:, 16:16 + H]
    al = jnp.where(al >= 0.0, al, 0.2 * al)
    ws = jnp.exp(al)
    r = r_ref[...]
    ws_bc = jnp.dot(ws, r, preferred_element_type=jnp.float32)
    den = jnp.sum(dp_ref[0], axis=0) + ws
    den_bc = jnp.dot(den, r, preferred_element_type=jnp.float32) + 1e-16
    o_ref[...] = (a + ws_bc * h_ref[...]) / den_bc + b_ref[...]


def _combine(p, dp, h, sd, bias, r):
    grid = N // BN
    return pl.pallas_call(
        _comb_body,
        grid=(grid,),
        in_specs=[
            pl.BlockSpec((1, BN, HC), lambda i: (i // BPH, i % BPH, 0)),
            pl.BlockSpec((1, NS, BN, H), lambda i: (i // BPH, 0, i % BPH, 0)),
            pl.BlockSpec((BN, HC), lambda i: (i, 0)),
            pl.BlockSpec((BN, HC), lambda i: (i, 0)),
            pl.BlockSpec((1, HC), lambda i: (0, 0)),
            pl.BlockSpec((H, HC), lambda i: (0, 0)),
        ],
        out_specs=pl.BlockSpec((BN, HC), lambda i: (i, 0)),
        out_shape=jax.ShapeDtypeStruct((N, HC), jnp.float32),
    )(p, dp, h, sd, bias, r)


def kernel(x, edge_index, W, att_src, att_dst, bias):
    src = edge_index[0]
    dst = edge_index[1]

    eye = jnp.eye(H, dtype=jnp.float32)
    asrc = jnp.reshape(eye[:, None, :] * att_src.reshape(H, C)[:, :, None],
                       (HC, H))
    adst = jnp.reshape(eye[:, None, :] * att_dst.reshape(H, C)[:, :, None],
                       (HC, H))
    amat = jnp.concatenate(
        [asrc, asrc, adst, adst,
         jnp.zeros((HC, HC - 4 * H), jnp.float32)], axis=1)

    a2mat = jnp.concatenate([adst[:, 0::2], adst[:, 1::2]], axis=1)
    h, sd, pd = _project(x, W, amat, a2mat, bn=1000)
    partial, denp = _sc_edges(src, dst, h, sd, pd.reshape(N * HP))

    r = jnp.repeat(jnp.eye(H, dtype=jnp.float32), C, axis=1)
    out = _combine(partial, denp.reshape(NC, NS, AROWS, H), h, sd,
                   bias.reshape(1, HC), r)
    return out
